# baseline (speedup 1.0000x reference)
import functools

import jax
import jax.numpy as jnp
from jax import lax
from jax.experimental import pallas as pl
from jax.experimental.pallas import tpu as pltpu
from jax.experimental.pallas import tpu_sc as plsc

N = 10000
D = 128
E = 160000
NC = 2
NS = 16
NW = NC * NS
EPW = E // NW
CH = 128
NFULL = EPW // CH
TAIL = EPW - NFULL * CH
RPW = 632
RPW_LAST = N - (NS - 1) * RPW

EPS_LN = 1e-5
T = 512
NPAD = 10240
NT = NPAD // T

CPT = 40
EPAD = NW * CPT * CH
R2D = EPAD // CH
NDUMP = 8


@functools.cache
def _build_sc_degrees():
  mesh = plsc.VectorSubcoreMesh(core_axis_name="c", subcore_axis_name="s",
                                num_cores=NC, num_subcores=NS)

  @functools.partial(
      pl.kernel,
      out_type=(jax.ShapeDtypeStruct((NW * N,), jnp.float32),
                jax.ShapeDtypeStruct((NW * N,), jnp.float32)),
      mesh=mesh,
      compiler_params=pltpu.CompilerParams(needs_layout_passes=False),
      scratch_types=[
          pltpu.VMEM((EPW + 16,), jnp.int32),
          pltpu.VMEM((EPW + 16,), jnp.int32),
          pltpu.VMEM((N,), jnp.float32),
          pltpu.VMEM((N,), jnp.float32),
      ],
  )
  def sc_degrees(ei, out_s, out_d, src_v, dst_v, hs_v, hd_v):
    c = lax.axis_index("c")
    s = lax.axis_index("s")
    wid = s * NC + c
    base = wid * EPW

    def zbody(i, carry):
        z = jnp.zeros((16,), jnp.float32)
        hs_v[pl.ds(i * 16, 16)] = z
        hd_v[pl.ds(i * 16, 16)] = z
        return carry

    lax.fori_loop(0, N // 16, zbody, 0)

    pltpu.sync_copy(ei.at[pl.ds(base, EPW)], src_v.at[pl.ds(0, EPW)])
    pltpu.sync_copy(ei.at[pl.ds(E + base, EPW)], dst_v.at[pl.ds(0, EPW)])
    zi = jnp.zeros((16,), jnp.int32)
    src_v[pl.ds(EPW, 16)] = zi
    dst_v[pl.ds(EPW, 16)] = zi

    ones = jnp.ones((16,), jnp.float32)

    def body(k, carry):
        plsc.addupdate_scatter(hs_v, [src_v[pl.ds(k * 16, 16)]], ones)
        plsc.addupdate_scatter(hd_v, [dst_v[pl.ds(k * 16, 16)]], ones)
        return carry

    nfull = EPW // 16
    lax.fori_loop(0, nfull, body, 0)
    rem = EPW - nfull * 16
    m = lax.iota(jnp.int32, 16) < rem
    plsc.addupdate_scatter(hs_v, [src_v[pl.ds(nfull * 16, 16)]], ones, mask=m)
    plsc.addupdate_scatter(hd_v, [dst_v[pl.ds(nfull * 16, 16)]], ones, mask=m)

    pltpu.sync_copy(hs_v, out_s.at[pl.ds(wid * N, N)])
    pltpu.sync_copy(hd_v, out_d.at[pl.ds(wid * N, N)])

  return sc_degrees


def _sc_degrees(ei):
    return _build_sc_degrees()(ei)


@functools.cache
def _build_sc_agg():
  mesh = plsc.VectorSubcoreMesh(core_axis_name="c", subcore_axis_name="s",
                                num_cores=NC, num_subcores=NS)

  @functools.partial(
      pl.kernel,
      out_type=jax.ShapeDtypeStruct((NC * N, D), jnp.float32),
      mesh=mesh,
      compiler_params=pltpu.CompilerParams(needs_layout_passes=False),
      scratch_types=[
          pltpu.VMEM_SHARED((N + NDUMP, D), jnp.float32),
          pltpu.VMEM((CPT, CH), jnp.int32),
          pltpu.VMEM((CPT, CH), jnp.int32),
          pltpu.VMEM((CH, D), jnp.float32),
          pltpu.VMEM((CH, D), jnp.float32),
          pltpu.SemaphoreType.DMA,
          pltpu.SemaphoreType.DMA,
          pltpu.SemaphoreType.DMA,
          pltpu.SemaphoreType.DMA,
      ],
  )
  def sc_agg(h_hbm, s2d, d2d, z_hbm, out, shared, sidx_all, didx_all,
             r0, r1, sg0, sg1, ss0, ss1):
    c = lax.axis_index("c")
    s = lax.axis_index("s")
    wid = s * NC + c
    rbase = wid * CPT

    pltpu.sync_copy(s2d.at[pl.ds(rbase, CPT)], sidx_all)
    pltpu.sync_copy(d2d.at[pl.ds(rbase, CPT)], didx_all)

    @pl.when(s < NS - 1)
    def _():
        pltpu.sync_copy(z_hbm.at[pl.ds(s * RPW, RPW)],
                        shared.at[pl.ds(s * RPW, RPW)])

    @pl.when(s == NS - 1)
    def _():
        pltpu.sync_copy(z_hbm.at[pl.ds((NS - 1) * RPW, RPW_LAST)],
                        shared.at[pl.ds((NS - 1) * RPW, RPW_LAST)])

    plsc.subcore_barrier()

    rows = (r0, r1)
    sg = (sg0, sg1)
    ss = (ss0, ss1)

    pltpu.async_copy(h_hbm.at[sidx_all.at[0]], r0, sg0)
    pltpu.async_copy(h_hbm.at[sidx_all.at[1]], r1, sg1)

    def body(g, carry):
        for b in range(2):
            k = g * 2 + b
            pltpu.make_async_copy(h_hbm.at[sidx_all.at[k]], rows[b],
                                  sg[b]).wait()
            pltpu.async_copy(rows[b], shared.at[didx_all.at[k]], ss[b],
                             add=True)
            pltpu.make_async_copy(rows[b], shared.at[pl.ds(0, CH)],
                                  ss[b]).wait()

            @pl.when(k + 2 < CPT)
            def _():
                pltpu.async_copy(h_hbm.at[sidx_all.at[k + 2]], rows[b],
                                 sg[b])
        return carry

    lax.fori_loop(0, CPT // 2, body, 0)

    plsc.subcore_barrier()

    @pl.when(s < NS - 1)
    def _():
        pltpu.sync_copy(shared.at[pl.ds(s * RPW, RPW)],
                        out.at[pl.ds(c * N + s * RPW, RPW)])

    @pl.when(s == NS - 1)
    def _():
        pltpu.sync_copy(shared.at[pl.ds((NS - 1) * RPW, RPW_LAST)],
                        out.at[pl.ds(c * N + (NS - 1) * RPW, RPW_LAST)])

  return sc_agg


def _sc_agg(h, s2d, d2d, zrows):
    return _build_sc_agg()(h, s2d, d2d, zrows)


@functools.cache
def _build_sc_gather2():
  mesh = plsc.VectorSubcoreMesh(core_axis_name="c", subcore_axis_name="s",
                                num_cores=NC, num_subcores=NS)

  @functools.partial(
      pl.kernel,
      out_type=(jax.ShapeDtypeStruct((EPAD, D), jnp.float32),
                jax.ShapeDtypeStruct((EPAD, D), jnp.float32)),
      mesh=mesh,
      compiler_params=pltpu.CompilerParams(needs_layout_passes=False),
      scratch_types=[
          pltpu.VMEM((CPT, CH), jnp.int32),
          pltpu.VMEM((CPT, CH), jnp.int32),
          pltpu.VMEM((CH, D), jnp.float32),
          pltpu.VMEM((CH, D), jnp.float32),
          pltpu.VMEM((CH, D), jnp.float32),
          pltpu.VMEM((CH, D), jnp.float32),
          pltpu.SemaphoreType.DMA,
          pltpu.SemaphoreType.DMA,
          pltpu.SemaphoreType.DMA,
          pltpu.SemaphoreType.DMA,
          pltpu.SemaphoreType.DMA,
          pltpu.SemaphoreType.DMA,
          pltpu.SemaphoreType.DMA,
          pltpu.SemaphoreType.DMA,
      ],
  )
  def sc_gather2(qe_hbm, s2d, d2d, rs_out, rd_out, sidx_all, didx_all,
                 rs0, rs1, rd0, rd1, sgs0, sgs1, sgd0, sgd1,
                 sws0, sws1, swd0, swd1):
    c = lax.axis_index("c")
    s = lax.axis_index("s")
    wid = s * NC + c
    rbase = wid * CPT
    obase = rbase * CH

    pltpu.sync_copy(s2d.at[pl.ds(rbase, CPT)], sidx_all)
    pltpu.sync_copy(d2d.at[pl.ds(rbase, CPT)], didx_all)

    rws = (rs0, rs1)
    rwd = (rd0, rd1)
    sgs = (sgs0, sgs1)
    sgd = (sgd0, sgd1)
    sws = (sws0, sws1)
    swd = (swd0, swd1)

    pltpu.async_copy(qe_hbm.at[sidx_all.at[0]], rs0, sgs0)
    pltpu.async_copy(qe_hbm.at[didx_all.at[0]], rd0, sgd0)
    pltpu.async_copy(qe_hbm.at[sidx_all.at[1]], rs1, sgs1)
    pltpu.async_copy(qe_hbm.at[didx_all.at[1]], rd1, sgd1)

    def body(g, carry):
        for b in range(2):
            k = g * 2 + b
            o = obase + k * CH
            pltpu.make_async_copy(qe_hbm.at[sidx_all.at[k]], rws[b],
                                  sgs[b]).wait()
            pltpu.async_copy(rws[b], rs_out.at[pl.ds(o, CH)], sws[b])
            pltpu.make_async_copy(qe_hbm.at[didx_all.at[k]], rwd[b],
                                  sgd[b]).wait()
            pltpu.async_copy(rwd[b], rd_out.at[pl.ds(o, CH)], swd[b])
            pltpu.make_async_copy(rws[b], rs_out.at[pl.ds(o, CH)],
                                  sws[b]).wait()
            pltpu.make_async_copy(rwd[b], rd_out.at[pl.ds(o, CH)],
                                  swd[b]).wait()

            @pl.when(k + 2 < CPT)
            def _():
                pltpu.async_copy(qe_hbm.at[sidx_all.at[k + 2]], rws[b],
                                 sgs[b])
                pltpu.async_copy(qe_hbm.at[didx_all.at[k + 2]], rwd[b],
                                 sgd[b])
        return carry

    lax.fori_loop(0, CPT // 2, body, 0)

  return sc_gather2


def _sc_gather2(qe, s2d, d2d):
    return _build_sc_gather2()(qe, s2d, d2d)


_RB = 1000


def _scales_body(feats_ref, hst_ref, hdt_ref, hs1_ref, so_ref, si_ref):
    od = jnp.sum(hst_ref[...], axis=1, keepdims=True)
    idg = jnp.sum(hdt_ref[...], axis=1, keepdims=True)
    so = lax.rsqrt(jnp.maximum(od, 1.0))
    si = lax.rsqrt(jnp.maximum(idg, 1.0))
    hs1_ref[...] = feats_ref[...] * so
    so_ref[...] = jnp.broadcast_to(so, (_RB, D))
    si_ref[...] = jnp.broadcast_to(si, (_RB, D))


def _tc_scales(feats, hst, hdt):
    return pl.pallas_call(
        _scales_body,
        grid=(N // _RB,),
        in_specs=[
            pl.BlockSpec((_RB, D), lambda i: (i, 0)),
            pl.BlockSpec((_RB, NW), lambda i: (i, 0)),
            pl.BlockSpec((_RB, NW), lambda i: (i, 0)),
        ],
        out_specs=[
            pl.BlockSpec((_RB, D), lambda i: (i, 0)),
            pl.BlockSpec((_RB, D), lambda i: (i, 0)),
            pl.BlockSpec((_RB, D), lambda i: (i, 0)),
        ],
        out_shape=[jax.ShapeDtypeStruct((N, D), jnp.float32)] * 3,
    )(feats, hst, hdt)


def _layer1_body(a0_ref, a1_ref, si_ref, so_ref, w_ref, b_ref, g_ref,
                 be_ref, h1_ref, hs2_ref):
    a = (a0_ref[...] + a1_ref[...]) * si_ref[...]
    z = lax.dot(a, w_ref[...], precision=lax.Precision.HIGHEST) + b_ref[...]
    h = jnp.maximum(z, 0.0)
    mu = jnp.mean(h, axis=1, keepdims=True)
    dlt = h - mu
    var = jnp.mean(dlt * dlt, axis=1, keepdims=True)
    h1 = dlt * lax.rsqrt(var + EPS_LN) * g_ref[...] + be_ref[...]
    h1_ref[...] = h1
    hs2_ref[...] = h1 * so_ref[...]


def _tc_layer1(a0, a1, si, so, w, b, g, be):
    row = pl.BlockSpec((_RB, D), lambda i: (i, 0))
    vec = pl.BlockSpec((1, D), lambda i: (0, 0))
    return pl.pallas_call(
        _layer1_body,
        grid=(N // _RB,),
        in_specs=[row, row, row, row,
                  pl.BlockSpec((D, D), lambda i: (0, 0)), vec, vec, vec],
        out_specs=[row, row],
        out_shape=[jax.ShapeDtypeStruct((N, D), jnp.float32)] * 2,
    )(a0, a1, si, so, w, b, g, be)


def _layer2_body(a0_ref, a1_ref, si_ref, w_ref, b_ref, dw1_ref, db1_ref,
                 dw2_ref, db2_ref, h2_ref, qe_ref, sse_ref, acc_ref):
    i = pl.program_id(0)

    @pl.when(i == 0)
    def _():
        acc_ref[0] = 0.0

    a = (a0_ref[...] + a1_ref[...]) * si_ref[...]
    h2 = jnp.maximum(
        lax.dot(a, w_ref[...], precision=lax.Precision.HIGHEST) + b_ref[...],
        0.0)
    qe = lax.dot(h2, dw1_ref[...], precision=lax.Precision.HIGHEST) + db1_ref[...]
    qn = lax.dot(h2, dw2_ref[...], precision=lax.Precision.HIGHEST) + db2_ref[...]
    h2_ref[...] = h2
    qe_ref[...] = qe
    r = h2 - qn
    acc_ref[0] += jnp.sum(r * r)

    @pl.when(i == N // _RB - 1)
    def _():
        sse_ref[0, 0] = acc_ref[0]


def _tc_layer2(a0, a1, si, w, b, dw1, db1, dw2, db2):
    row = pl.BlockSpec((_RB, D), lambda i: (i, 0))
    vec = pl.BlockSpec((1, D), lambda i: (0, 0))
    mat = pl.BlockSpec((D, D), lambda i: (0, 0))
    return pl.pallas_call(
        _layer2_body,
        grid=(N // _RB,),
        in_specs=[row, row, row, mat, vec, mat, vec, mat, vec],
        out_specs=[row, row,
                   pl.BlockSpec((1, 1), lambda i: (0, 0),
                                memory_space=pltpu.SMEM)],
        out_shape=[jax.ShapeDtypeStruct((N, D), jnp.float32),
                   jax.ShapeDtypeStruct((N, D), jnp.float32),
                   jax.ShapeDtypeStruct((1, 1), jnp.float32)],
        scratch_shapes=[pltpu.SMEM((1,), jnp.float32)],
    )(a0, a1, si, w, b, dw1, db1, dw2, db2)


_EB = 2048


def _softplus(x):
    return jnp.maximum(x, 0.0) + jnp.log1p(jnp.exp(-jnp.abs(x)))


def _corr_body(rs_ref, rd_ref, s_ref, d_ref, c1_ref, c2_ref, ne_ref, acc_ref):
    k = pl.program_id(0)

    @pl.when(k == 0)
    def _():
        acc_ref[0] = 0.0
        acc_ref[1] = 0.0
        acc_ref[2] = 0.0

    p = jnp.sum(rs_ref[...] * rd_ref[...], axis=1, keepdims=True)
    valid = (s_ref[...] < d_ref[...]).astype(jnp.float32)
    acc_ref[0] += jnp.sum(valid * _softplus(p))
    acc_ref[1] += jnp.sum(valid * _softplus(-p))
    acc_ref[2] += jnp.sum(valid)

    @pl.when(k == EPAD // _EB - 1)
    def _():
        c1_ref[0, 0] = acc_ref[0]
        c2_ref[0, 0] = acc_ref[1]
        ne_ref[0, 0] = acc_ref[2]


def _tc_corr(rs, rd, scol, dcol):
    scal = pl.BlockSpec((1, 1), lambda k: (0, 0), memory_space=pltpu.SMEM)
    return pl.pallas_call(
        _corr_body,
        grid=(EPAD // _EB,),
        in_specs=[
            pl.BlockSpec((_EB, D), lambda k: (k, 0)),
            pl.BlockSpec((_EB, D), lambda k: (k, 0)),
            pl.BlockSpec((_EB, 1), lambda k: (k, 0)),
            pl.BlockSpec((_EB, 1), lambda k: (k, 0)),
        ],
        out_specs=[scal, scal, scal],
        out_shape=[jax.ShapeDtypeStruct((1, 1), jnp.float32)] * 3,
        scratch_shapes=[pltpu.SMEM((3,), jnp.float32)],
    )(rs, rd, scol, dcol)


def _loss_body(qi_ref, qj_ref, out_ref, acc_ref):
    i = pl.program_id(0)
    j = pl.program_id(1)

    @pl.when((i == 0) & (j == 0))
    def _():
        acc_ref[0] = 0.0

    @pl.when(j >= i)
    def _():
        p = lax.dot_general(qi_ref[...], qj_ref[...],
                            (((1,), (1,)), ((), ())),
                            precision=lax.Precision.HIGHEST)
        gr = i * T + lax.broadcasted_iota(jnp.int32, (T, T), 0)
        gc = j * T + lax.broadcasted_iota(jnp.int32, (T, T), 1)
        mask = (gr < gc) & (gc < N)
        acc_ref[0] += jnp.sum(jnp.where(mask, _softplus(p), 0.0))

    @pl.when((i == NT - 1) & (j == NT - 1))
    def _():
        out_ref[0, 0] = acc_ref[0]


def _tc_loss(qe_pad):
    return pl.pallas_call(
        _loss_body,
        grid=(NT, NT),
        in_specs=[
            pl.BlockSpec((T, D), lambda i, j: (i, 0)),
            pl.BlockSpec((T, D), lambda i, j: (j, 0)),
        ],
        out_specs=pl.BlockSpec((1, 1), lambda i, j: (0, 0),
                               memory_space=pltpu.SMEM),
        out_shape=jax.ShapeDtypeStruct((1, 1), jnp.float32),
        scratch_shapes=[pltpu.SMEM((1,), jnp.float32)],
    )(qe_pad, qe_pad)


def kernel(feats, edge_index, W1, b1, W2, b2, gamma, beta, dW1, db1, dW2,
           db2):
    ei = edge_index.reshape(-1)
    src_pad = jnp.pad(edge_index[0], (0, EPAD - E))
    dst_pad = jnp.pad(edge_index[1], (0, EPAD - E), constant_values=N)
    s2d = src_pad.reshape(R2D, CH)
    d2d = dst_pad.reshape(R2D, CH)
    scol = src_pad.reshape(EPAD, 1)
    dcol = jnp.pad(edge_index[1], (0, EPAD - E)).reshape(EPAD, 1)
    zrows = jnp.zeros((N, D), jnp.float32)
    b1r = b1.reshape(1, D)
    b2r = b2.reshape(1, D)
    db1r = db1.reshape(1, D)
    db2r = db2.reshape(1, D)
    gr = gamma.reshape(1, D)
    ber = beta.reshape(1, D)

    hs_p, hd_p = _sc_degrees(ei)
    hst = hs_p.reshape(NW, N).T
    hdt = hd_p.reshape(NW, N).T

    hs1, so_b, si_b = _tc_scales(feats, hst, hdt)

    aggp1 = _sc_agg(hs1, s2d, d2d, zrows)
    h1, hs2 = _tc_layer1(aggp1[:N], aggp1[N:], si_b, so_b, W1, b1r, gr, ber)

    aggp2 = _sc_agg(hs2, s2d, d2d, zrows)
    h2, qe, sse = _tc_layer2(aggp2[:N], aggp2[N:], si_b, W2, b2r, dW1, db1r,
                             dW2, db2r)

    qe_pad = jnp.pad(qe, ((0, NPAD - N), (0, 0)))
    rs, rd = _sc_gather2(qe_pad, s2d, d2d)
    c1, c2, ne = _tc_corr(rs, rd, scol, dcol)

    s_sp = _tc_loss(qe_pad)

    nef = ne[0, 0]
    pos_weight = (N * N / 2.0 - nef) / (nef + 1e-6)
    edge_sum = s_sp[0, 0] - c1[0, 0] + pos_weight * c2[0, 0]
    edge_loss = edge_sum / (N * (N - 1) / 2.0)
    feat_loss = sse[0, 0] / (N * D)
    loss = feat_loss + 100.0 * edge_loss

    return (h1, h2, qe, h2, loss)


# speedup vs baseline: 1.3379x; 1.3379x over previous
import functools

import jax
import jax.numpy as jnp
from jax import lax
from jax.experimental import pallas as pl
from jax.experimental.pallas import tpu as pltpu
from jax.experimental.pallas import tpu_sc as plsc

N = 10000
D = 128
E = 160000
NC = 2
NS = 16
NW = NC * NS
EPW = E // NW
CH = 128
NFULL = EPW // CH
TAIL = EPW - NFULL * CH
RPW = 632
RPW_LAST = N - (NS - 1) * RPW

EPS_LN = 1e-5
T = 512
NPAD = 10240
NT = NPAD // T

CPT = 40
EPAD = NW * CPT * CH
R2D = EPAD // CH
NDUMP = 128


@functools.cache
def _build_sc_degrees():
  mesh = plsc.VectorSubcoreMesh(core_axis_name="c", subcore_axis_name="s",
                                num_cores=NC, num_subcores=NS)

  @functools.partial(
      pl.kernel,
      out_type=(jax.ShapeDtypeStruct((NW * N,), jnp.float32),
                jax.ShapeDtypeStruct((NW * N,), jnp.float32)),
      mesh=mesh,
      compiler_params=pltpu.CompilerParams(needs_layout_passes=False),
      scratch_types=[
          pltpu.VMEM((EPW + 16,), jnp.int32),
          pltpu.VMEM((EPW + 16,), jnp.int32),
          pltpu.VMEM((N,), jnp.float32),
          pltpu.VMEM((N,), jnp.float32),
      ],
  )
  def sc_degrees(ei, out_s, out_d, src_v, dst_v, hs_v, hd_v):
    c = lax.axis_index("c")
    s = lax.axis_index("s")
    wid = s * NC + c
    base = wid * EPW

    def zbody(i, carry):
        z = jnp.zeros((16,), jnp.float32)
        hs_v[pl.ds(i * 16, 16)] = z
        hd_v[pl.ds(i * 16, 16)] = z
        return carry

    lax.fori_loop(0, N // 16, zbody, 0)

    pltpu.sync_copy(ei.at[pl.ds(base, EPW)], src_v.at[pl.ds(0, EPW)])
    pltpu.sync_copy(ei.at[pl.ds(E + base, EPW)], dst_v.at[pl.ds(0, EPW)])
    zi = jnp.zeros((16,), jnp.int32)
    src_v[pl.ds(EPW, 16)] = zi
    dst_v[pl.ds(EPW, 16)] = zi

    ones = jnp.ones((16,), jnp.float32)

    def body(k, carry):
        plsc.addupdate_scatter(hs_v, [src_v[pl.ds(k * 16, 16)]], ones)
        plsc.addupdate_scatter(hd_v, [dst_v[pl.ds(k * 16, 16)]], ones)
        return carry

    nfull = EPW // 16
    lax.fori_loop(0, nfull, body, 0)
    rem = EPW - nfull * 16
    m = lax.iota(jnp.int32, 16) < rem
    plsc.addupdate_scatter(hs_v, [src_v[pl.ds(nfull * 16, 16)]], ones, mask=m)
    plsc.addupdate_scatter(hd_v, [dst_v[pl.ds(nfull * 16, 16)]], ones, mask=m)

    pltpu.sync_copy(hs_v, out_s.at[pl.ds(wid * N, N)])
    pltpu.sync_copy(hd_v, out_d.at[pl.ds(wid * N, N)])

  return sc_degrees


def _sc_degrees(ei):
    return _build_sc_degrees()(ei)


@functools.cache
def _build_sc_agg():
  mesh = plsc.VectorSubcoreMesh(core_axis_name="c", subcore_axis_name="s",
                                num_cores=NC, num_subcores=NS)

  @functools.partial(
      pl.kernel,
      out_type=jax.ShapeDtypeStruct((NC * N, D), jnp.float32),
      mesh=mesh,
      compiler_params=pltpu.CompilerParams(needs_layout_passes=False),
      scratch_types=[
          pltpu.VMEM_SHARED((N + NDUMP, D), jnp.float32),
          pltpu.VMEM((CPT, CH), jnp.int32),
          pltpu.VMEM((CPT, CH), jnp.int32),
          pltpu.VMEM((CH, D), jnp.float32),
          pltpu.VMEM((CH, D), jnp.float32),
          pltpu.SemaphoreType.DMA,
          pltpu.SemaphoreType.DMA,
          pltpu.SemaphoreType.DMA,
          pltpu.SemaphoreType.DMA,
      ],
  )
  def sc_agg(h_hbm, s2d, d2d, z_hbm, out, shared, sidx_all, didx_all,
             r0, r1, sg0, sg1, ss0, ss1):
    c = lax.axis_index("c")
    s = lax.axis_index("s")
    wid = s * NC + c
    rbase = wid * CPT

    pltpu.sync_copy(s2d.at[pl.ds(rbase, CPT)], sidx_all)
    pltpu.sync_copy(d2d.at[pl.ds(rbase, CPT)], didx_all)

    @pl.when(s < NS - 1)
    def _():
        pltpu.sync_copy(z_hbm.at[pl.ds(s * RPW, RPW)],
                        shared.at[pl.ds(s * RPW, RPW)])

    @pl.when(s == NS - 1)
    def _():
        pltpu.sync_copy(z_hbm.at[pl.ds((NS - 1) * RPW, RPW_LAST)],
                        shared.at[pl.ds((NS - 1) * RPW, RPW_LAST)])

    plsc.subcore_barrier()

    rows = (r0, r1)
    sg = (sg0, sg1)
    ss = (ss0, ss1)

    pltpu.async_copy(h_hbm.at[sidx_all.at[0]], r0, sg0)
    pltpu.async_copy(h_hbm.at[sidx_all.at[1]], r1, sg1)

    def body(g, carry):
        for b in range(2):
            k = g * 2 + b
            pltpu.make_async_copy(h_hbm.at[sidx_all.at[k]], rows[b],
                                  sg[b]).wait()
            pltpu.async_copy(rows[b], shared.at[didx_all.at[k]], ss[b],
                             add=True)
            pltpu.make_async_copy(rows[b], shared.at[pl.ds(0, CH)],
                                  ss[b]).wait()

            @pl.when(k + 2 < CPT)
            def _():
                pltpu.async_copy(h_hbm.at[sidx_all.at[k + 2]], rows[b],
                                 sg[b])
        return carry

    lax.fori_loop(0, CPT // 2, body, 0)

    plsc.subcore_barrier()

    @pl.when(s < NS - 1)
    def _():
        pltpu.sync_copy(shared.at[pl.ds(s * RPW, RPW)],
                        out.at[pl.ds(c * N + s * RPW, RPW)])

    @pl.when(s == NS - 1)
    def _():
        pltpu.sync_copy(shared.at[pl.ds((NS - 1) * RPW, RPW_LAST)],
                        out.at[pl.ds(c * N + (NS - 1) * RPW, RPW_LAST)])

  return sc_agg


def _sc_agg(h, s2d, d2d, zrows):
    return _build_sc_agg()(h, s2d, d2d, zrows)


@functools.cache
def _build_sc_gather2():
  mesh = plsc.VectorSubcoreMesh(core_axis_name="c", subcore_axis_name="s",
                                num_cores=NC, num_subcores=NS)

  @functools.partial(
      pl.kernel,
      out_type=(jax.ShapeDtypeStruct((EPAD, D), jnp.float32),
                jax.ShapeDtypeStruct((EPAD, D), jnp.float32)),
      mesh=mesh,
      compiler_params=pltpu.CompilerParams(needs_layout_passes=False),
      scratch_types=[
          pltpu.VMEM((CPT, CH), jnp.int32),
          pltpu.VMEM((CPT, CH), jnp.int32),
          pltpu.VMEM((CH, D), jnp.float32),
          pltpu.VMEM((CH, D), jnp.float32),
          pltpu.VMEM((CH, D), jnp.float32),
          pltpu.VMEM((CH, D), jnp.float32),
          pltpu.SemaphoreType.DMA,
          pltpu.SemaphoreType.DMA,
          pltpu.SemaphoreType.DMA,
          pltpu.SemaphoreType.DMA,
          pltpu.SemaphoreType.DMA,
          pltpu.SemaphoreType.DMA,
          pltpu.SemaphoreType.DMA,
          pltpu.SemaphoreType.DMA,
      ],
  )
  def sc_gather2(qe_hbm, s2d, d2d, rs_out, rd_out, sidx_all, didx_all,
                 rs0, rs1, rd0, rd1, sgs0, sgs1, sgd0, sgd1,
                 sws0, sws1, swd0, swd1):
    c = lax.axis_index("c")
    s = lax.axis_index("s")
    wid = s * NC + c
    rbase = wid * CPT
    obase = rbase * CH

    pltpu.sync_copy(s2d.at[pl.ds(rbase, CPT)], sidx_all)
    pltpu.sync_copy(d2d.at[pl.ds(rbase, CPT)], didx_all)

    rws = (rs0, rs1)
    rwd = (rd0, rd1)
    sgs = (sgs0, sgs1)
    sgd = (sgd0, sgd1)
    sws = (sws0, sws1)
    swd = (swd0, swd1)

    pltpu.async_copy(qe_hbm.at[sidx_all.at[0]], rs0, sgs0)
    pltpu.async_copy(qe_hbm.at[didx_all.at[0]], rd0, sgd0)
    pltpu.async_copy(qe_hbm.at[sidx_all.at[1]], rs1, sgs1)
    pltpu.async_copy(qe_hbm.at[didx_all.at[1]], rd1, sgd1)

    def body(g, carry):
        for b in range(2):
            k = g * 2 + b
            o = obase + k * CH
            pltpu.make_async_copy(qe_hbm.at[sidx_all.at[k]], rws[b],
                                  sgs[b]).wait()
            pltpu.async_copy(rws[b], rs_out.at[pl.ds(o, CH)], sws[b])
            pltpu.make_async_copy(qe_hbm.at[didx_all.at[k]], rwd[b],
                                  sgd[b]).wait()
            pltpu.async_copy(rwd[b], rd_out.at[pl.ds(o, CH)], swd[b])
            pltpu.make_async_copy(rws[b], rs_out.at[pl.ds(o, CH)],
                                  sws[b]).wait()
            pltpu.make_async_copy(rwd[b], rd_out.at[pl.ds(o, CH)],
                                  swd[b]).wait()

            @pl.when(k + 2 < CPT)
            def _():
                pltpu.async_copy(qe_hbm.at[sidx_all.at[k + 2]], rws[b],
                                 sgs[b])
                pltpu.async_copy(qe_hbm.at[didx_all.at[k + 2]], rwd[b],
                                 sgd[b])
        return carry

    lax.fori_loop(0, CPT // 2, body, 0)

  return sc_gather2


def _sc_gather2(qe, s2d, d2d):
    return _build_sc_gather2()(qe, s2d, d2d)


_RB = 1000


def _scales_body(feats_ref, hst_ref, hdt_ref, hs1_ref, so_ref, si_ref):
    od = jnp.sum(hst_ref[...], axis=1, keepdims=True)
    idg = jnp.sum(hdt_ref[...], axis=1, keepdims=True)
    so = lax.rsqrt(jnp.maximum(od, 1.0))
    si = lax.rsqrt(jnp.maximum(idg, 1.0))
    hs1_ref[...] = feats_ref[...] * so
    so_ref[...] = jnp.broadcast_to(so, (_RB, D))
    si_ref[...] = jnp.broadcast_to(si, (_RB, D))


def _tc_scales(feats, hst, hdt):
    return pl.pallas_call(
        _scales_body,
        grid=(N // _RB,),
        in_specs=[
            pl.BlockSpec((_RB, D), lambda i: (i, 0)),
            pl.BlockSpec((_RB, NW), lambda i: (i, 0)),
            pl.BlockSpec((_RB, NW), lambda i: (i, 0)),
        ],
        out_specs=[
            pl.BlockSpec((_RB, D), lambda i: (i, 0)),
            pl.BlockSpec((_RB, D), lambda i: (i, 0)),
            pl.BlockSpec((_RB, D), lambda i: (i, 0)),
        ],
        out_shape=[jax.ShapeDtypeStruct((N, D), jnp.float32)] * 3,
    )(feats, hst, hdt)


def _layer1_body(a0_ref, a1_ref, si_ref, so_ref, w_ref, b_ref, g_ref,
                 be_ref, h1_ref, hs2_ref):
    a = (a0_ref[...] + a1_ref[...]) * si_ref[...]
    z = lax.dot(a, w_ref[...], precision=lax.Precision.HIGHEST) + b_ref[...]
    h = jnp.maximum(z, 0.0)
    mu = jnp.mean(h, axis=1, keepdims=True)
    dlt = h - mu
    var = jnp.mean(dlt * dlt, axis=1, keepdims=True)
    h1 = dlt * lax.rsqrt(var + EPS_LN) * g_ref[...] + be_ref[...]
    h1_ref[...] = h1
    hs2_ref[...] = h1 * so_ref[...]


def _tc_layer1(a0, a1, si, so, w, b, g, be):
    row = pl.BlockSpec((_RB, D), lambda i: (i, 0))
    vec = pl.BlockSpec((1, D), lambda i: (0, 0))
    return pl.pallas_call(
        _layer1_body,
        grid=(N // _RB,),
        in_specs=[row, row, row, row,
                  pl.BlockSpec((D, D), lambda i: (0, 0)), vec, vec, vec],
        out_specs=[row, row],
        out_shape=[jax.ShapeDtypeStruct((N, D), jnp.float32)] * 2,
    )(a0, a1, si, so, w, b, g, be)


def _layer2_body(a0_ref, a1_ref, si_ref, w_ref, b_ref, dw1_ref, db1_ref,
                 dw2_ref, db2_ref, h2_ref, qe_ref, sse_ref, acc_ref):
    i = pl.program_id(0)

    @pl.when(i == 0)
    def _():
        acc_ref[0] = 0.0

    a = (a0_ref[...] + a1_ref[...]) * si_ref[...]
    h2 = jnp.maximum(
        lax.dot(a, w_ref[...], precision=lax.Precision.HIGHEST) + b_ref[...],
        0.0)
    qe = lax.dot(h2, dw1_ref[...], precision=lax.Precision.HIGHEST) + db1_ref[...]
    qn = lax.dot(h2, dw2_ref[...], precision=lax.Precision.HIGHEST) + db2_ref[...]
    h2_ref[...] = h2
    qe_ref[...] = qe
    r = h2 - qn
    acc_ref[0] += jnp.sum(r * r)

    @pl.when(i == N // _RB - 1)
    def _():
        sse_ref[0, 0] = acc_ref[0]


def _tc_layer2(a0, a1, si, w, b, dw1, db1, dw2, db2):
    row = pl.BlockSpec((_RB, D), lambda i: (i, 0))
    vec = pl.BlockSpec((1, D), lambda i: (0, 0))
    mat = pl.BlockSpec((D, D), lambda i: (0, 0))
    return pl.pallas_call(
        _layer2_body,
        grid=(N // _RB,),
        in_specs=[row, row, row, mat, vec, mat, vec, mat, vec],
        out_specs=[row, row,
                   pl.BlockSpec((1, 1), lambda i: (0, 0),
                                memory_space=pltpu.SMEM)],
        out_shape=[jax.ShapeDtypeStruct((N, D), jnp.float32),
                   jax.ShapeDtypeStruct((N, D), jnp.float32),
                   jax.ShapeDtypeStruct((1, 1), jnp.float32)],
        scratch_shapes=[pltpu.SMEM((1,), jnp.float32)],
    )(a0, a1, si, w, b, dw1, db1, dw2, db2)


_EB = 2048


def _softplus(x):
    return jnp.maximum(x, 0.0) + jnp.log1p(jnp.exp(-jnp.abs(x)))


def _corr_body(rs_ref, rd_ref, s_ref, d_ref, c1_ref, c2_ref, ne_ref, acc_ref):
    k = pl.program_id(0)

    @pl.when(k == 0)
    def _():
        acc_ref[0] = 0.0
        acc_ref[1] = 0.0
        acc_ref[2] = 0.0

    p = jnp.sum(rs_ref[...] * rd_ref[...], axis=1, keepdims=True)
    valid = (s_ref[...] < d_ref[...]).astype(jnp.float32)
    acc_ref[0] += jnp.sum(valid * _softplus(p))
    acc_ref[1] += jnp.sum(valid * _softplus(-p))
    acc_ref[2] += jnp.sum(valid)

    @pl.when(k == EPAD // _EB - 1)
    def _():
        c1_ref[0, 0] = acc_ref[0]
        c2_ref[0, 0] = acc_ref[1]
        ne_ref[0, 0] = acc_ref[2]


def _tc_corr(rs, rd, scol, dcol):
    scal = pl.BlockSpec((1, 1), lambda k: (0, 0), memory_space=pltpu.SMEM)
    return pl.pallas_call(
        _corr_body,
        grid=(EPAD // _EB,),
        in_specs=[
            pl.BlockSpec((_EB, D), lambda k: (k, 0)),
            pl.BlockSpec((_EB, D), lambda k: (k, 0)),
            pl.BlockSpec((_EB, 1), lambda k: (k, 0)),
            pl.BlockSpec((_EB, 1), lambda k: (k, 0)),
        ],
        out_specs=[scal, scal, scal],
        out_shape=[jax.ShapeDtypeStruct((1, 1), jnp.float32)] * 3,
        scratch_shapes=[pltpu.SMEM((3,), jnp.float32)],
    )(rs, rd, scol, dcol)


def _loss_body(qi_ref, qj_ref, out_ref, acc_ref):
    i = pl.program_id(0)
    j = pl.program_id(1)

    @pl.when((i == 0) & (j == 0))
    def _():
        acc_ref[0] = 0.0

    @pl.when(j >= i)
    def _():
        p = lax.dot_general(qi_ref[...], qj_ref[...],
                            (((1,), (1,)), ((), ())),
                            precision=lax.Precision.HIGHEST)
        gr = i * T + lax.broadcasted_iota(jnp.int32, (T, T), 0)
        gc = j * T + lax.broadcasted_iota(jnp.int32, (T, T), 1)
        mask = (gr < gc) & (gc < N)
        acc_ref[0] += jnp.sum(jnp.where(mask, _softplus(p), 0.0))

    @pl.when((i == NT - 1) & (j == NT - 1))
    def _():
        out_ref[0, 0] = acc_ref[0]


def _tc_loss(qe_pad):
    return pl.pallas_call(
        _loss_body,
        grid=(NT, NT),
        in_specs=[
            pl.BlockSpec((T, D), lambda i, j: (i, 0)),
            pl.BlockSpec((T, D), lambda i, j: (j, 0)),
        ],
        out_specs=pl.BlockSpec((1, 1), lambda i, j: (0, 0),
                               memory_space=pltpu.SMEM),
        out_shape=jax.ShapeDtypeStruct((1, 1), jnp.float32),
        scratch_shapes=[pltpu.SMEM((1,), jnp.float32)],
    )(qe_pad, qe_pad)


def kernel(feats, edge_index, W1, b1, W2, b2, gamma, beta, dW1, db1, dW2,
           db2):
    ei = edge_index.reshape(-1)
    npd = EPAD - E
    pad_ar = jnp.arange(npd, dtype=jnp.int32)
    src_pad = jnp.concatenate([edge_index[0], (pad_ar * 8) % N])
    dst_pad = jnp.concatenate([edge_index[1], N + pad_ar % NDUMP])
    s2d = src_pad.reshape(R2D, CH)
    d2d = dst_pad.reshape(R2D, CH)
    scol = src_pad.reshape(EPAD, 1)
    dcol = jnp.pad(edge_index[1], (0, EPAD - E)).reshape(EPAD, 1)
    zrows = jnp.zeros((N, D), jnp.float32)
    b1r = b1.reshape(1, D)
    b2r = b2.reshape(1, D)
    db1r = db1.reshape(1, D)
    db2r = db2.reshape(1, D)
    gr = gamma.reshape(1, D)
    ber = beta.reshape(1, D)

    hs_p, hd_p = _sc_degrees(ei)
    hst = hs_p.reshape(NW, N).T
    hdt = hd_p.reshape(NW, N).T

    hs1, so_b, si_b = _tc_scales(feats, hst, hdt)

    aggp1 = _sc_agg(hs1, s2d, d2d, zrows)
    h1, hs2 = _tc_layer1(aggp1[:N], aggp1[N:], si_b, so_b, W1, b1r, gr, ber)

    aggp2 = _sc_agg(hs2, s2d, d2d, zrows)
    h2, qe, sse = _tc_layer2(aggp2[:N], aggp2[N:], si_b, W2, b2r, dW1, db1r,
                             dW2, db2r)

    qe_pad = jnp.pad(qe, ((0, NPAD - N), (0, 0)))
    rs, rd = _sc_gather2(qe_pad, s2d, d2d)
    c1, c2, ne = _tc_corr(rs, rd, scol, dcol)

    s_sp = _tc_loss(qe_pad)

    nef = ne[0, 0]
    pos_weight = (N * N / 2.0 - nef) / (nef + 1e-6)
    edge_sum = s_sp[0, 0] - c1[0, 0] + pos_weight * c2[0, 0]
    edge_loss = edge_sum / (N * (N - 1) / 2.0)
    feat_loss = sse[0, 0] / (N * D)
    loss = feat_loss + 100.0 * edge_loss

    return (h1, h2, qe, h2, loss)


# speedup vs baseline: 1.9440x; 1.4531x over previous
import functools

import jax
import jax.numpy as jnp
from jax import lax
from jax.experimental import pallas as pl
from jax.experimental.pallas import tpu as pltpu
from jax.experimental.pallas import tpu_sc as plsc

N = 10000
D = 128
E = 160000
NC = 2
NS = 16
NW = NC * NS
EPW = E // NW
CH = 128
NFULL = EPW // CH
TAIL = EPW - NFULL * CH
RPW = 632
RPW_LAST = N - (NS - 1) * RPW

EPS_LN = 1e-5
T = 1024
NPAD = 10240
NT = NPAD // T
NTILES = NT * (NT + 1) // 2

CPT = 40
EPAD = NW * CPT * CH
R2D = EPAD // CH
NDUMP = 128


@functools.cache
def _build_sc_degrees():
  mesh = plsc.VectorSubcoreMesh(core_axis_name="c", subcore_axis_name="s",
                                num_cores=NC, num_subcores=NS)

  @functools.partial(
      pl.kernel,
      out_type=(jax.ShapeDtypeStruct((NW * N,), jnp.float32),
                jax.ShapeDtypeStruct((NW * N,), jnp.float32)),
      mesh=mesh,
      compiler_params=pltpu.CompilerParams(needs_layout_passes=False),
      scratch_types=[
          pltpu.VMEM((EPW + 16,), jnp.int32),
          pltpu.VMEM((EPW + 16,), jnp.int32),
          pltpu.VMEM((N,), jnp.float32),
          pltpu.VMEM((N,), jnp.float32),
      ],
  )
  def sc_degrees(ei, out_s, out_d, src_v, dst_v, hs_v, hd_v):
    c = lax.axis_index("c")
    s = lax.axis_index("s")
    wid = s * NC + c
    base = wid * EPW

    def zbody(i, carry):
        z = jnp.zeros((16,), jnp.float32)
        hs_v[pl.ds(i * 16, 16)] = z
        hd_v[pl.ds(i * 16, 16)] = z
        return carry

    lax.fori_loop(0, N // 16, zbody, 0)

    pltpu.sync_copy(ei.at[pl.ds(base, EPW)], src_v.at[pl.ds(0, EPW)])
    pltpu.sync_copy(ei.at[pl.ds(E + base, EPW)], dst_v.at[pl.ds(0, EPW)])
    zi = jnp.zeros((16,), jnp.int32)
    src_v[pl.ds(EPW, 16)] = zi
    dst_v[pl.ds(EPW, 16)] = zi

    ones = jnp.ones((16,), jnp.float32)

    def body(k, carry):
        plsc.addupdate_scatter(hs_v, [src_v[pl.ds(k * 16, 16)]], ones)
        plsc.addupdate_scatter(hd_v, [dst_v[pl.ds(k * 16, 16)]], ones)
        return carry

    nfull = EPW // 16
    lax.fori_loop(0, nfull, body, 0)
    rem = EPW - nfull * 16
    m = lax.iota(jnp.int32, 16) < rem
    plsc.addupdate_scatter(hs_v, [src_v[pl.ds(nfull * 16, 16)]], ones, mask=m)
    plsc.addupdate_scatter(hd_v, [dst_v[pl.ds(nfull * 16, 16)]], ones, mask=m)

    pltpu.sync_copy(hs_v, out_s.at[pl.ds(wid * N, N)])
    pltpu.sync_copy(hd_v, out_d.at[pl.ds(wid * N, N)])

  return sc_degrees


def _sc_degrees(ei):
    return _build_sc_degrees()(ei)


@functools.cache
def _build_sc_agg():
  mesh = plsc.VectorSubcoreMesh(core_axis_name="c", subcore_axis_name="s",
                                num_cores=NC, num_subcores=NS)

  @functools.partial(
      pl.kernel,
      out_type=jax.ShapeDtypeStruct((NC * N, D), jnp.float32),
      mesh=mesh,
      compiler_params=pltpu.CompilerParams(needs_layout_passes=False),
      scratch_types=[
          pltpu.VMEM_SHARED((N + NDUMP, D), jnp.float32),
          pltpu.VMEM((CPT, CH), jnp.int32),
          pltpu.VMEM((CPT, CH), jnp.int32),
          pltpu.VMEM((CH, D), jnp.float32),
          pltpu.VMEM((CH, D), jnp.float32),
          pltpu.SemaphoreType.DMA,
          pltpu.SemaphoreType.DMA,
          pltpu.SemaphoreType.DMA,
          pltpu.SemaphoreType.DMA,
      ],
  )
  def sc_agg(h_hbm, s2d, d2d, z_hbm, out, shared, sidx_all, didx_all,
             r0, r1, sg0, sg1, ss0, ss1):
    c = lax.axis_index("c")
    s = lax.axis_index("s")
    wid = s * NC + c
    rbase = wid * CPT

    pltpu.sync_copy(s2d.at[pl.ds(rbase, CPT)], sidx_all)
    pltpu.sync_copy(d2d.at[pl.ds(rbase, CPT)], didx_all)

    @pl.when(s < NS - 1)
    def _():
        pltpu.sync_copy(z_hbm.at[pl.ds(s * RPW, RPW)],
                        shared.at[pl.ds(s * RPW, RPW)])

    @pl.when(s == NS - 1)
    def _():
        pltpu.sync_copy(z_hbm.at[pl.ds((NS - 1) * RPW, RPW_LAST)],
                        shared.at[pl.ds((NS - 1) * RPW, RPW_LAST)])

    plsc.subcore_barrier()

    rows = (r0, r1)
    sg = (sg0, sg1)
    ss = (ss0, ss1)

    pltpu.async_copy(h_hbm.at[sidx_all.at[0]], r0, sg0)
    pltpu.async_copy(h_hbm.at[sidx_all.at[1]], r1, sg1)

    def body(g, carry):
        for b in range(2):
            k = g * 2 + b
            pltpu.make_async_copy(h_hbm.at[sidx_all.at[k]], rows[b],
                                  sg[b]).wait()
            pltpu.async_copy(rows[b], shared.at[didx_all.at[k]], ss[b],
                             add=True)
            pltpu.make_async_copy(rows[b], shared.at[pl.ds(0, CH)],
                                  ss[b]).wait()

            @pl.when(k + 2 < CPT)
            def _():
                pltpu.async_copy(h_hbm.at[sidx_all.at[k + 2]], rows[b],
                                 sg[b])
        return carry

    lax.fori_loop(0, CPT // 2, body, 0)

    plsc.subcore_barrier()

    @pl.when(s < NS - 1)
    def _():
        pltpu.sync_copy(shared.at[pl.ds(s * RPW, RPW)],
                        out.at[pl.ds(c * N + s * RPW, RPW)])

    @pl.when(s == NS - 1)
    def _():
        pltpu.sync_copy(shared.at[pl.ds((NS - 1) * RPW, RPW_LAST)],
                        out.at[pl.ds(c * N + (NS - 1) * RPW, RPW_LAST)])

  return sc_agg


def _sc_agg(h, s2d, d2d, zrows):
    return _build_sc_agg()(h, s2d, d2d, zrows)


@functools.cache
def _build_sc_gather2():
  mesh = plsc.VectorSubcoreMesh(core_axis_name="c", subcore_axis_name="s",
                                num_cores=NC, num_subcores=NS)

  @functools.partial(
      pl.kernel,
      out_type=(jax.ShapeDtypeStruct((EPAD, D), jnp.float32),
                jax.ShapeDtypeStruct((EPAD, D), jnp.float32)),
      mesh=mesh,
      compiler_params=pltpu.CompilerParams(needs_layout_passes=False),
      scratch_types=[
          pltpu.VMEM((CPT, CH), jnp.int32),
          pltpu.VMEM((CPT, CH), jnp.int32),
          pltpu.VMEM((CH, D), jnp.float32),
          pltpu.VMEM((CH, D), jnp.float32),
          pltpu.VMEM((CH, D), jnp.float32),
          pltpu.VMEM((CH, D), jnp.float32),
          pltpu.SemaphoreType.DMA,
          pltpu.SemaphoreType.DMA,
          pltpu.SemaphoreType.DMA,
          pltpu.SemaphoreType.DMA,
          pltpu.SemaphoreType.DMA,
          pltpu.SemaphoreType.DMA,
          pltpu.SemaphoreType.DMA,
          pltpu.SemaphoreType.DMA,
      ],
  )
  def sc_gather2(qe_hbm, s2d, d2d, rs_out, rd_out, sidx_all, didx_all,
                 rs0, rs1, rd0, rd1, sgs0, sgs1, sgd0, sgd1,
                 sws0, sws1, swd0, swd1):
    c = lax.axis_index("c")
    s = lax.axis_index("s")
    wid = s * NC + c
    rbase = wid * CPT
    obase = rbase * CH

    pltpu.sync_copy(s2d.at[pl.ds(rbase, CPT)], sidx_all)
    pltpu.sync_copy(d2d.at[pl.ds(rbase, CPT)], didx_all)

    rws = (rs0, rs1)
    rwd = (rd0, rd1)
    sgs = (sgs0, sgs1)
    sgd = (sgd0, sgd1)
    sws = (sws0, sws1)
    swd = (swd0, swd1)

    pltpu.async_copy(qe_hbm.at[sidx_all.at[0]], rs0, sgs0)
    pltpu.async_copy(qe_hbm.at[didx_all.at[0]], rd0, sgd0)
    pltpu.async_copy(qe_hbm.at[sidx_all.at[1]], rs1, sgs1)
    pltpu.async_copy(qe_hbm.at[didx_all.at[1]], rd1, sgd1)

    def body(g, carry):
        for b in range(2):
            k = g * 2 + b
            o = obase + k * CH
            pltpu.make_async_copy(qe_hbm.at[sidx_all.at[k]], rws[b],
                                  sgs[b]).wait()
            pltpu.async_copy(rws[b], rs_out.at[pl.ds(o, CH)], sws[b])
            pltpu.make_async_copy(qe_hbm.at[didx_all.at[k]], rwd[b],
                                  sgd[b]).wait()
            pltpu.async_copy(rwd[b], rd_out.at[pl.ds(o, CH)], swd[b])
            pltpu.make_async_copy(rws[b], rs_out.at[pl.ds(o, CH)],
                                  sws[b]).wait()
            pltpu.make_async_copy(rwd[b], rd_out.at[pl.ds(o, CH)],
                                  swd[b]).wait()

            @pl.when(k + 2 < CPT)
            def _():
                pltpu.async_copy(qe_hbm.at[sidx_all.at[k + 2]], rws[b],
                                 sgs[b])
                pltpu.async_copy(qe_hbm.at[didx_all.at[k + 2]], rwd[b],
                                 sgd[b])
        return carry

    lax.fori_loop(0, CPT // 2, body, 0)

  return sc_gather2


def _sc_gather2(qe, s2d, d2d):
    return _build_sc_gather2()(qe, s2d, d2d)


_RB = 1000


def _scales_body(feats_ref, hst_ref, hdt_ref, hs1_ref, so_ref, si_ref):
    od = jnp.sum(hst_ref[...], axis=1, keepdims=True)
    idg = jnp.sum(hdt_ref[...], axis=1, keepdims=True)
    so = lax.rsqrt(jnp.maximum(od, 1.0))
    si = lax.rsqrt(jnp.maximum(idg, 1.0))
    hs1_ref[...] = feats_ref[...] * so
    so_ref[...] = jnp.broadcast_to(so, (_RB, D))
    si_ref[...] = jnp.broadcast_to(si, (_RB, D))


def _tc_scales(feats, hst, hdt):
    return pl.pallas_call(
        _scales_body,
        grid=(N // _RB,),
        in_specs=[
            pl.BlockSpec((_RB, D), lambda i: (i, 0)),
            pl.BlockSpec((_RB, NW), lambda i: (i, 0)),
            pl.BlockSpec((_RB, NW), lambda i: (i, 0)),
        ],
        out_specs=[
            pl.BlockSpec((_RB, D), lambda i: (i, 0)),
            pl.BlockSpec((_RB, D), lambda i: (i, 0)),
            pl.BlockSpec((_RB, D), lambda i: (i, 0)),
        ],
        out_shape=[jax.ShapeDtypeStruct((N, D), jnp.float32)] * 3,
    )(feats, hst, hdt)


def _layer1_body(a0_ref, a1_ref, si_ref, so_ref, w_ref, b_ref, g_ref,
                 be_ref, h1_ref, hs2_ref):
    a = (a0_ref[...] + a1_ref[...]) * si_ref[...]
    z = lax.dot(a, w_ref[...], precision=lax.Precision.HIGHEST) + b_ref[...]
    h = jnp.maximum(z, 0.0)
    mu = jnp.mean(h, axis=1, keepdims=True)
    dlt = h - mu
    var = jnp.mean(dlt * dlt, axis=1, keepdims=True)
    h1 = dlt * lax.rsqrt(var + EPS_LN) * g_ref[...] + be_ref[...]
    h1_ref[...] = h1
    hs2_ref[...] = h1 * so_ref[...]


def _tc_layer1(a0, a1, si, so, w, b, g, be):
    row = pl.BlockSpec((_RB, D), lambda i: (i, 0))
    vec = pl.BlockSpec((1, D), lambda i: (0, 0))
    return pl.pallas_call(
        _layer1_body,
        grid=(N // _RB,),
        in_specs=[row, row, row, row,
                  pl.BlockSpec((D, D), lambda i: (0, 0)), vec, vec, vec],
        out_specs=[row, row],
        out_shape=[jax.ShapeDtypeStruct((N, D), jnp.float32)] * 2,
    )(a0, a1, si, so, w, b, g, be)


def _layer2_body(a0_ref, a1_ref, si_ref, w_ref, b_ref, dw1_ref, db1_ref,
                 dw2_ref, db2_ref, h2_ref, qe_ref, sse_ref, acc_ref):
    i = pl.program_id(0)

    @pl.when(i == 0)
    def _():
        acc_ref[0] = 0.0

    a = (a0_ref[...] + a1_ref[...]) * si_ref[...]
    h2 = jnp.maximum(
        lax.dot(a, w_ref[...], precision=lax.Precision.HIGHEST) + b_ref[...],
        0.0)
    qe = lax.dot(h2, dw1_ref[...], precision=lax.Precision.HIGHEST) + db1_ref[...]
    qn = lax.dot(h2, dw2_ref[...], precision=lax.Precision.HIGHEST) + db2_ref[...]
    h2_ref[...] = h2
    qe_ref[...] = qe
    r = h2 - qn
    acc_ref[0] += jnp.sum(r * r)

    @pl.when(i == N // _RB - 1)
    def _():
        sse_ref[0, 0] = acc_ref[0]


def _tc_layer2(a0, a1, si, w, b, dw1, db1, dw2, db2):
    row = pl.BlockSpec((_RB, D), lambda i: (i, 0))
    vec = pl.BlockSpec((1, D), lambda i: (0, 0))
    mat = pl.BlockSpec((D, D), lambda i: (0, 0))
    return pl.pallas_call(
        _layer2_body,
        grid=(N // _RB,),
        in_specs=[row, row, row, mat, vec, mat, vec, mat, vec],
        out_specs=[row, row,
                   pl.BlockSpec((1, 1), lambda i: (0, 0),
                                memory_space=pltpu.SMEM)],
        out_shape=[jax.ShapeDtypeStruct((N, D), jnp.float32),
                   jax.ShapeDtypeStruct((N, D), jnp.float32),
                   jax.ShapeDtypeStruct((1, 1), jnp.float32)],
        scratch_shapes=[pltpu.SMEM((1,), jnp.float32)],
    )(a0, a1, si, w, b, dw1, db1, dw2, db2)


_EB = 2048
_EBR = _EB // CH


def _softplus(x):
    return jnp.maximum(x, 0.0) + jnp.log1p(jnp.exp(-jnp.abs(x)))


def _corr_body(rs_ref, rd_ref, s_ref, d_ref, c1_ref, c2_ref, ne_ref, acc_ref):
    k = pl.program_id(0)

    @pl.when(k == 0)
    def _():
        acc_ref[0] = 0.0
        acc_ref[1] = 0.0
        acc_ref[2] = 0.0

    prod = rs_ref[...].astype(jnp.float32) * rd_ref[...].astype(jnp.float32)
    p = jnp.sum(prod, axis=1).reshape(_EBR, CH)
    vf = (s_ref[...] < d_ref[...]).astype(jnp.float32)
    acc_ref[0] += jnp.sum(vf * _softplus(p))
    acc_ref[1] += jnp.sum(vf * _softplus(-p))
    acc_ref[2] += jnp.sum(vf)

    @pl.when(k == EPAD // _EB - 1)
    def _():
        c1_ref[0, 0] = acc_ref[0]
        c2_ref[0, 0] = acc_ref[1]
        ne_ref[0, 0] = acc_ref[2]


def _tc_corr(rs, rd, s2d, d2d):
    scal = pl.BlockSpec((1, 1), lambda k: (0, 0), memory_space=pltpu.SMEM)
    return pl.pallas_call(
        _corr_body,
        grid=(EPAD // _EB,),
        in_specs=[
            pl.BlockSpec((_EB, D), lambda k: (k, 0)),
            pl.BlockSpec((_EB, D), lambda k: (k, 0)),
            pl.BlockSpec((_EBR, CH), lambda k: (k, 0)),
            pl.BlockSpec((_EBR, CH), lambda k: (k, 0)),
        ],
        out_specs=[scal, scal, scal],
        out_shape=[jax.ShapeDtypeStruct((1, 1), jnp.float32)] * 3,
        scratch_shapes=[pltpu.SMEM((3,), jnp.float32)],
    )(rs, rd, s2d, d2d)


def _loss_body(iof_ref, jof_ref, qi_ref, qj_ref, out_ref, acc_ref):
    t = pl.program_id(0)
    i = iof_ref[t]
    j = jof_ref[t]

    @pl.when(t == 0)
    def _():
        acc_ref[0] = 0.0

    p = lax.dot_general(qi_ref[...], qj_ref[...],
                        (((1,), (1,)), ((), ())),
                        precision=lax.Precision.HIGHEST)
    sp = _softplus(p)

    @pl.when((j > i) & (j < NT - 1))
    def _():
        acc_ref[0] += jnp.sum(sp)

    @pl.when((j > i) & (j == NT - 1))
    def _():
        gc = j * T + lax.broadcasted_iota(jnp.int32, (T, T), 1)
        acc_ref[0] += jnp.sum(jnp.where(gc < N, sp, 0.0))

    @pl.when(j == i)
    def _():
        rl = lax.broadcasted_iota(jnp.int32, (T, T), 0)
        cl = lax.broadcasted_iota(jnp.int32, (T, T), 1)
        gc = j * T + cl
        acc_ref[0] += jnp.sum(jnp.where((rl < cl) & (gc < N), sp, 0.0))

    @pl.when(t == NTILES - 1)
    def _():
        out_ref[0, 0] = acc_ref[0]


def _tc_loss(qe_pad, iof, jof):
    grid_spec = pltpu.PrefetchScalarGridSpec(
        num_scalar_prefetch=2,
        grid=(NTILES,),
        in_specs=[
            pl.BlockSpec((T, D), lambda t, io, jo: (io[t], 0)),
            pl.BlockSpec((T, D), lambda t, io, jo: (jo[t], 0)),
        ],
        out_specs=pl.BlockSpec((1, 1), lambda t, io, jo: (0, 0),
                               memory_space=pltpu.SMEM),
        scratch_shapes=[pltpu.SMEM((1,), jnp.float32)],
    )
    return pl.pallas_call(
        _loss_body,
        grid_spec=grid_spec,
        out_shape=jax.ShapeDtypeStruct((1, 1), jnp.float32),
    )(iof, jof, qe_pad, qe_pad)


def kernel(feats, edge_index, W1, b1, W2, b2, gamma, beta, dW1, db1, dW2,
           db2):
    ei = edge_index.reshape(-1)
    npd = EPAD - E
    pad_ar = jnp.arange(npd, dtype=jnp.int32)
    src_pad = jnp.concatenate([edge_index[0], (pad_ar * 8) % N])
    dst_pad = jnp.concatenate([edge_index[1], N + pad_ar % NDUMP])
    s2d = src_pad.reshape(R2D, CH)
    d2d = dst_pad.reshape(R2D, CH)
    dcol2d = jnp.pad(edge_index[1], (0, EPAD - E)).reshape(R2D, CH)
    zrows = jnp.zeros((N, D), jnp.float32)
    b1r = b1.reshape(1, D)
    b2r = b2.reshape(1, D)
    db1r = db1.reshape(1, D)
    db2r = db2.reshape(1, D)
    gr = gamma.reshape(1, D)
    ber = beta.reshape(1, D)

    hs_p, hd_p = _sc_degrees(ei)
    hst = hs_p.reshape(NW, N).T
    hdt = hd_p.reshape(NW, N).T

    hs1, so_b, si_b = _tc_scales(feats, hst, hdt)

    aggp1 = _sc_agg(hs1, s2d, d2d, zrows)
    h1, hs2 = _tc_layer1(aggp1[:N], aggp1[N:], si_b, so_b, W1, b1r, gr, ber)

    aggp2 = _sc_agg(hs2, s2d, d2d, zrows)
    h2, qe, sse = _tc_layer2(aggp2[:N], aggp2[N:], si_b, W2, b2r, dW1, db1r,
                             dW2, db2r)

    qe_pad = jnp.pad(qe, ((0, NPAD - N), (0, 0)))
    rs, rd = _sc_gather2(qe_pad, s2d, d2d)
    c1, c2, ne = _tc_corr(rs, rd, s2d, dcol2d)

    iof = jnp.asarray([i for i in range(NT) for _ in range(i, NT)],
                      dtype=jnp.int32)
    jof = jnp.asarray([j for i in range(NT) for j in range(i, NT)],
                      dtype=jnp.int32)
    s_sp = _tc_loss(qe_pad, iof, jof)

    nef = ne[0, 0]
    pos_weight = (N * N / 2.0 - nef) / (nef + 1e-6)
    edge_sum = s_sp[0, 0] - c1[0, 0] + pos_weight * c2[0, 0]
    edge_loss = edge_sum / (N * (N - 1) / 2.0)
    feat_loss = sse[0, 0] / (N * D)
    loss = feat_loss + 100.0 * edge_loss

    return (h1, h2, qe, h2, loss)
